# baseline (speedup 1.0000x reference)
import functools

import jax
import jax.numpy as jnp
from jax import lax
from jax.experimental import pallas as pl
from jax.experimental.pallas import tpu as pltpu
from jax.experimental.pallas import tpu_sc as plsc

B = 16384
K = 16
NUM_CORES = 2
NUM_SUBCORES = 16
NW = NUM_CORES * NUM_SUBCORES
BPW = B // NW
GRP = 8
NGRP = BPW // GRP
BLK = 2048


def _gather_body(col, xt_hbm, t_hbm, out_hbm, idxv, slabs, rows, sem):
    wid = lax.axis_index("s") * NUM_CORES + lax.axis_index("c")
    base = wid * BPW

    def idx_chunk(c, carry):
        off = c * 128
        pltpu.sync_copy(xt_hbm.at[col, pl.ds(base + off, 128)],
                        idxv.at[pl.ds(off, 128)])
        return carry

    lax.fori_loop(0, BPW // 128, idx_chunk, 0)

    lanes = lax.iota(jnp.int32, 16)

    def group_body(g, carry):
        cols16a = idxv[pl.ds(g * GRP, 16)]
        for j in range(GRP):
            cj = cols16a[j]
            bj = pl.multiple_of((cj // 128) * 128, 128)
            for kt in range(2):
                pltpu.async_copy(
                    t_hbm.at[pl.ds(kt * 8, 8), pl.ds(bj, 128)],
                    slabs.at[j, kt], sem)
        for _ in range(2 * GRP):
            pltpu.make_async_copy(t_hbm.at[pl.ds(0, 8), pl.ds(0, 128)],
                                  slabs.at[0, 0], sem).wait()
        ktv = lanes // 8
        krv = lax.rem(lanes, 8)
        for j in range(GRP):
            cj = cols16a[j]
            jv = jnp.full((16,), 0, jnp.int32) + j
            cm = jnp.full((16,), 0, jnp.int32) + lax.rem(cj, 128)
            r = plsc.load_gather(slabs, [jv, ktv, krv, cm])
            row = g * GRP + j
            plsc.store_scatter(
                rows, [jnp.full((16,), 0, jnp.int32) + row, lanes], r)
        return carry

    lax.fori_loop(0, NGRP, group_body, 0)

    def wb_chunk(c, carry):
        off = c * 32
        pltpu.sync_copy(rows.at[pl.ds(off, 32)],
                        out_hbm.at[pl.ds(base + off, 32)])
        return carry

    lax.fori_loop(0, BPW // 32, wb_chunk, 0)


def _sc_gather(xt, table_t, col):
    mesh = plsc.VectorSubcoreMesh(core_axis_name="c", subcore_axis_name="s")
    return pl.kernel(
        functools.partial(_gather_body, col),
        out_type=jax.ShapeDtypeStruct((B, K), jnp.float32),
        mesh=mesh,
        compiler_params=pltpu.CompilerParams(needs_layout_passes=False),
        scratch_types=[
            pltpu.VMEM((BPW + 16,), jnp.int32),
            pltpu.VMEM((GRP, 2, 8, 128), jnp.float32),
            pltpu.VMEM((BPW, K), jnp.float32),
            pltpu.SemaphoreType.DMA,
        ],
    )(xt, table_t)


def _tc_body(u_ref, v_ref, dot_ref, cat_ref):
    u = u_ref[...]
    v = v_ref[...]
    dot_ref[...] = jnp.sum(u * v, axis=1)
    cat_ref[:, :K] = u
    cat_ref[:, K:] = v


def kernel(x, W, H):
    xt = x.T
    u = _sc_gather(xt, W.T, 0)
    v = _sc_gather(xt, H.T, 1)
    dot, cat = pl.pallas_call(
        _tc_body,
        grid=(B // BLK,),
        in_specs=[pl.BlockSpec((BLK, K), lambda i: (i, 0)),
                  pl.BlockSpec((BLK, K), lambda i: (i, 0))],
        out_specs=[pl.BlockSpec((BLK,), lambda i: (i,)),
                   pl.BlockSpec((BLK, 2 * K), lambda i: (i, 0))],
        out_shape=[jax.ShapeDtypeStruct((B,), jnp.float32),
                   jax.ShapeDtypeStruct((B, 2 * K), jnp.float32)],
    )(u, v)
    return dot, cat


# speedup vs baseline: 1.0458x; 1.0458x over previous
import functools

import jax
import jax.numpy as jnp
from jax import lax
from jax.experimental import pallas as pl
from jax.experimental.pallas import tpu as pltpu
from jax.experimental.pallas import tpu_sc as plsc

B = 16384
K = 16
NUM_CORES = 2
NUM_SUBCORES = 16
NW = NUM_CORES * NUM_SUBCORES
BPW = B // NW
GRP = 8
NGRP = BPW // GRP
BLK = 2048


def _gather_body(col, xt_hbm, t_hbm, out_hbm, idxv, slabs, rows, sem):
    wid = lax.axis_index("s") * NUM_CORES + lax.axis_index("c")
    base = wid * BPW

    def idx_chunk(c, carry):
        off = c * 128
        pltpu.sync_copy(xt_hbm.at[col, pl.ds(base + off, 128)],
                        idxv.at[pl.ds(off, 128)])
        return carry

    lax.fori_loop(0, BPW // 128, idx_chunk, 0)

    lanes = lax.iota(jnp.int32, 16)

    def fire(g):
        cols16 = idxv[pl.ds(g * GRP, 16)]
        slot0 = lax.rem(g, 2) * GRP
        for j in range(GRP):
            cj = cols16[j]
            bj = pl.multiple_of((cj // 128) * 128, 128)
            for kt in range(2):
                pltpu.async_copy(
                    t_hbm.at[pl.ds(kt * 8, 8), pl.ds(bj, 128)],
                    slabs.at[slot0 + j, kt], sem)

    fire(jnp.int32(0))

    def group_body(g, carry):
        for _ in range(2 * GRP):
            pltpu.make_async_copy(t_hbm.at[pl.ds(0, 8), pl.ds(0, 128)],
                                  slabs.at[0, 0], sem).wait()

        @pl.when(g < NGRP - 1)
        def _():
            fire(g + 1)

        cols16 = idxv[pl.ds(g * GRP, 16)]
        slot0 = lax.rem(g, 2) * GRP
        ktv = lanes // 8
        krv = lax.rem(lanes, 8)
        for j in range(GRP):
            cj = cols16[j]
            jv = jnp.full((16,), 0, jnp.int32) + slot0 + j
            cm = jnp.full((16,), 0, jnp.int32) + lax.rem(cj, 128)
            r = plsc.load_gather(slabs, [jv, ktv, krv, cm])
            row = g * GRP + j
            plsc.store_scatter(
                rows, [jnp.full((16,), 0, jnp.int32) + row, lanes], r)
        return carry

    lax.fori_loop(0, NGRP, group_body, 0)

    def wb_chunk(c, carry):
        off = c * 32
        pltpu.sync_copy(rows.at[pl.ds(off, 32)],
                        out_hbm.at[pl.ds(base + off, 32)])
        return carry

    lax.fori_loop(0, BPW // 32, wb_chunk, 0)


def _sc_gather(xt, table_t, col):
    mesh = plsc.VectorSubcoreMesh(core_axis_name="c", subcore_axis_name="s")
    return pl.kernel(
        functools.partial(_gather_body, col),
        out_type=jax.ShapeDtypeStruct((B, K), jnp.float32),
        mesh=mesh,
        compiler_params=pltpu.CompilerParams(needs_layout_passes=False),
        scratch_types=[
            pltpu.VMEM((BPW + 16,), jnp.int32),
            pltpu.VMEM((2 * GRP, 2, 8, 128), jnp.float32),
            pltpu.VMEM((BPW, K), jnp.float32),
            pltpu.SemaphoreType.DMA,
        ],
    )(xt, table_t)


def _tc_body(u_ref, v_ref, dot_ref, cat_ref):
    u = u_ref[...]
    v = v_ref[...]
    dot_ref[...] = jnp.sum(u * v, axis=1)
    cat_ref[:, :K] = u
    cat_ref[:, K:] = v


def kernel(x, W, H):
    xt = x.T
    u = _sc_gather(xt, W.T, 0)
    v = _sc_gather(xt, H.T, 1)
    dot, cat = pl.pallas_call(
        _tc_body,
        grid=(B // BLK,),
        in_specs=[pl.BlockSpec((BLK, K), lambda i: (i, 0)),
                  pl.BlockSpec((BLK, K), lambda i: (i, 0))],
        out_specs=[pl.BlockSpec((BLK,), lambda i: (i,)),
                   pl.BlockSpec((BLK, 2 * K), lambda i: (i, 0))],
        out_shape=[jax.ShapeDtypeStruct((B,), jnp.float32),
                   jax.ShapeDtypeStruct((B, 2 * K), jnp.float32)],
    )(u, v)
    return dot, cat
